# speedup vs baseline: 1.3104x; 1.3104x over previous
import jax
import jax.numpy as jnp
from jax import lax
from jax.experimental import pallas as pl
from jax.experimental.pallas import tpu as pltpu
from jax.experimental.pallas import tpu_sc as plsc

B = 4
S = 4096
D = 128
NC = 2
NS = 16
NW = NC * NS
SW = S // NW
ROWS = B * SW
LANES = 16


def _emb_kernel(ids_hbm, tok_hbm, pos_hbm, out_hbm, idx_v, rows_v, pos_v,
                gsem, psem):
    wid = lax.axis_index("s") * NC + lax.axis_index("c")
    base = wid * SW

    for b in range(B):
        pltpu.sync_copy(ids_hbm.at[b, pl.ds(base, SW)],
                        idx_v.at[pl.ds(b * SW, SW)])

    pos_cp = pltpu.async_copy(pos_hbm.at[pl.ds(base, SW)], pos_v, psem)
    gat_cp = pltpu.async_copy(tok_hbm.at[idx_v], rows_v, gsem)
    pos_cp.wait()
    gat_cp.wait()

    def add_body(j, carry):
        for d in range(D // LANES):
            sl = pl.ds(d * LANES, LANES)
            p = pos_v[j, sl]
            for b in range(B):
                r = b * SW + j
                rows_v[r, sl] += p
        return carry

    lax.fori_loop(0, SW, add_body, 0)

    for b in range(B):
        pltpu.sync_copy(rows_v.at[pl.ds(b * SW, SW)],
                        out_hbm.at[b, pl.ds(base, SW), :])


@jax.jit
def _emb(input_ids, token_table, pos_table):
    mesh = plsc.VectorSubcoreMesh(core_axis_name="c", subcore_axis_name="s")
    return pl.kernel(
        _emb_kernel,
        mesh=mesh,
        out_type=jax.ShapeDtypeStruct((B, S, D), jnp.float32),
        scratch_types=[
            pltpu.VMEM((ROWS,), jnp.int32),
            pltpu.VMEM((ROWS, D), jnp.float32),
            pltpu.VMEM((SW, D), jnp.float32),
            pltpu.SemaphoreType.DMA,
            pltpu.SemaphoreType.DMA,
        ],
    )(input_ids, token_table, pos_table)


def kernel(input_ids, token_table, pos_table):
    return _emb(input_ids, token_table, pos_table)


# speedup vs baseline: 1.4345x; 1.0947x over previous
import jax
import jax.numpy as jnp
from jax import lax
from jax.experimental import pallas as pl
from jax.experimental.pallas import tpu as pltpu
from jax.experimental.pallas import tpu_sc as plsc

B = 4
S = 4096
D = 128
NC = 2
NS = 16
NW = NC * NS
SW = S // NW
ROWS = B * SW


def _emb_kernel(ids_hbm, tok_hbm, pos_hbm, out_hbm, idx_v, rows_v,
                isem, psem, gsem, ssem):
    wid = lax.axis_index("s") * NC + lax.axis_index("c")
    base = wid * SW

    icps, pcps, gcps, scps = [], [], [], []
    for b in range(B):
        sl = pl.ds(b * SW, SW)
        icps.append(pltpu.async_copy(ids_hbm.at[b, pl.ds(base, SW)],
                                     idx_v.at[sl], isem.at[b]))
        pcps.append(pltpu.async_copy(pos_hbm.at[pl.ds(base, SW)],
                                     rows_v.at[sl], psem.at[b]))
    for b in range(B):
        sl = pl.ds(b * SW, SW)
        icps[b].wait()
        pcps[b].wait()
        gcps.append(pltpu.async_copy(tok_hbm.at[idx_v.at[sl]],
                                     rows_v.at[sl], gsem.at[b], add=True))
    for b in range(B):
        sl = pl.ds(b * SW, SW)
        gcps[b].wait()
        scps.append(pltpu.async_copy(rows_v.at[sl],
                                     out_hbm.at[b, pl.ds(base, SW), :],
                                     ssem.at[b]))
    for b in range(B):
        scps[b].wait()


@jax.jit
def _emb(input_ids, token_table, pos_table):
    mesh = plsc.VectorSubcoreMesh(core_axis_name="c", subcore_axis_name="s")
    return pl.kernel(
        _emb_kernel,
        mesh=mesh,
        out_type=jax.ShapeDtypeStruct((B, S, D), jnp.float32),
        scratch_types=[
            pltpu.VMEM((ROWS,), jnp.int32),
            pltpu.VMEM((ROWS, D), jnp.float32),
            pltpu.SemaphoreType.DMA((B,)),
            pltpu.SemaphoreType.DMA((B,)),
            pltpu.SemaphoreType.DMA((B,)),
            pltpu.SemaphoreType.DMA((B,)),
        ],
    )(input_ids, token_table, pos_table)


def kernel(input_ids, token_table, pos_table):
    return _emb(input_ids, token_table, pos_table)


# speedup vs baseline: 1.4770x; 1.0296x over previous
import jax
import jax.numpy as jnp
from jax import lax
from jax.experimental import pallas as pl
from jax.experimental.pallas import tpu as pltpu
from jax.experimental.pallas import tpu_sc as plsc

B = 4
S = 4096
D = 128
NC = 2
NS = 16
NW = NC * NS
SW = S // NW
ROWS = B * SW
LANES = 16


def _emb_kernel(ids_hbm, tok_hbm, pos_hbm, out_hbm, idx_v, rows_v, pos_v,
                isem, psem, gsem, ssem):
    wid = lax.axis_index("s") * NC + lax.axis_index("c")
    base = wid * SW

    icps = []
    for b in range(B):
        icps.append(pltpu.async_copy(ids_hbm.at[b, pl.ds(base, SW)],
                                     idx_v.at[pl.ds(b * SW, SW)],
                                     isem.at[b]))
    pcp = pltpu.async_copy(pos_hbm.at[pl.ds(base, SW)], pos_v, psem)
    pcp.wait()

    gcps = []
    for b in range(B):
        def copy_body(j, carry, b=b):
            for d in range(D // LANES):
                sl = pl.ds(d * LANES, LANES)
                rows_v[b * SW + j, sl] = pos_v[j, sl]
            return carry

        lax.fori_loop(0, SW, copy_body, 0)
        icps[b].wait()
        sl = pl.ds(b * SW, SW)
        gcps.append(pltpu.async_copy(tok_hbm.at[idx_v.at[sl]],
                                     rows_v.at[sl], gsem.at[b], add=True))

    scps = []
    for b in range(B):
        gcps[b].wait()
        scps.append(pltpu.async_copy(rows_v.at[pl.ds(b * SW, SW)],
                                     out_hbm.at[b, pl.ds(base, SW), :],
                                     ssem.at[b]))
    for b in range(B):
        scps[b].wait()


@jax.jit
def _emb(input_ids, token_table, pos_table):
    mesh = plsc.VectorSubcoreMesh(core_axis_name="c", subcore_axis_name="s")
    return pl.kernel(
        _emb_kernel,
        mesh=mesh,
        out_type=jax.ShapeDtypeStruct((B, S, D), jnp.float32),
        scratch_types=[
            pltpu.VMEM((ROWS,), jnp.int32),
            pltpu.VMEM((ROWS, D), jnp.float32),
            pltpu.VMEM((SW, D), jnp.float32),
            pltpu.SemaphoreType.DMA((B,)),
            pltpu.SemaphoreType.DMA,
            pltpu.SemaphoreType.DMA((B,)),
            pltpu.SemaphoreType.DMA((B,)),
        ],
    )(input_ids, token_table, pos_table)


def kernel(input_ids, token_table, pos_table):
    return _emb(input_ids, token_table, pos_table)
